# Initial kernel scaffold; baseline (speedup 1.0000x reference)
import jax
import jax.numpy as jnp
from jax.experimental import pallas as pl


def kernel(features, edge_index, W):
    raise NotImplementedError("write your pallas kernel here")


# speedup vs baseline: 3.6467x; 3.6467x over previous
import functools

import jax
import jax.numpy as jnp
from jax import lax
from jax.experimental import pallas as pl
from jax.experimental.pallas import tpu as pltpu
from jax.experimental.pallas import tpu_sc as plsc

N = 10000
E = 320000
D = 128

NC = 2
NS = 16
NW = NC * NS
NPAD = 10240
EC = 128
NCHUNK = 79
EPW = EC * NCHUNK
EPAD = NW * EPW
RPT = NPAD // NS

_sc_mesh = plsc.VectorSubcoreMesh(
    core_axis_name="c", subcore_axis_name="s", num_cores=NC, num_subcores=NS)


@functools.partial(
    pl.kernel,
    out_type=jax.ShapeDtypeStruct((NC * NPAD,), jnp.float32),
    mesh=_sc_mesh,
    scratch_types=[
        pltpu.VMEM((EC,), jnp.int32),
        pltpu.VMEM((EC,), jnp.float32),
        pltpu.VMEM_SHARED((NPAD,), jnp.float32),
    ],
)
def _sc_degs(dst_hbm, zer_hbm, ones_hbm, out_hbm, idx_v, ones_v, acc_sh):
    c = lax.axis_index("c")
    s = lax.axis_index("s")
    wid = s * NC + c
    pltpu.sync_copy(zer_hbm, acc_sh.at[pl.ds(s * RPT, RPT)])
    pltpu.sync_copy(ones_hbm, ones_v)
    plsc.subcore_barrier()
    def body(i, carry):
        base = pl.multiple_of(wid * EPW + i * EC, EC)
        pltpu.sync_copy(dst_hbm.at[pl.ds(base, EC)], idx_v)
        pltpu.sync_copy(ones_v, acc_sh.at[idx_v], add=True)
        return carry
    lax.fori_loop(0, NCHUNK, body, 0)
    plsc.subcore_barrier()
    pltpu.sync_copy(acc_sh.at[pl.ds(s * RPT, RPT)],
                    out_hbm.at[pl.ds(c * NPAD + s * RPT, RPT)])


@functools.partial(
    pl.kernel,
    out_type=jax.ShapeDtypeStruct((NC * NPAD, D), jnp.float32),
    mesh=_sc_mesh,
    scratch_types=[
        pltpu.VMEM((EC,), jnp.int32),
        pltpu.VMEM((EC,), jnp.int32),
        pltpu.VMEM((EC, D), jnp.float32),
        pltpu.VMEM_SHARED((NPAD, D), jnp.float32),
        pltpu.SemaphoreType.DMA,
    ],
)
def _sc_prop(g_hbm, src_hbm, dst_hbm, zrows_hbm, out_hbm,
             sidx_v, didx_v, rows_v, acc_sh, sem):
    c = lax.axis_index("c")
    s = lax.axis_index("s")
    wid = s * NC + c
    pltpu.sync_copy(zrows_hbm, acc_sh.at[pl.ds(s * RPT, RPT)])
    plsc.subcore_barrier()
    def body(i, carry):
        base = pl.multiple_of(wid * EPW + i * EC, EC)
        pltpu.sync_copy(src_hbm.at[pl.ds(base, EC)], sidx_v)
        pltpu.sync_copy(dst_hbm.at[pl.ds(base, EC)], didx_v)
        pltpu.async_copy(g_hbm.at[sidx_v], rows_v, sem).wait()
        pltpu.sync_copy(rows_v, acc_sh.at[didx_v], add=True)
        return carry
    lax.fori_loop(0, NCHUNK, body, 0)
    plsc.subcore_barrier()
    pltpu.sync_copy(acc_sh.at[pl.ds(s * RPT, RPT)],
                    out_hbm.at[pl.ds(c * NPAD + s * RPT, RPT)])


_BM = 1280


def _tc_prep_body(x_ref, w_ref, d0_ref, d1_ref, g0_ref, ri_ref):
    deg = jnp.maximum(d0_ref[...] + d1_ref[...], 1.0)
    norm = lax.rsqrt(deg)
    h0 = lax.dot_general(x_ref[...], w_ref[...],
                         (((1,), (1,)), ((), ())),
                         preferred_element_type=jnp.float32)
    g0_ref[...] = h0 * norm
    ri_ref[...] = h0 * (norm * norm)


_tc_prep = pl.pallas_call(
    _tc_prep_body,
    grid=(NPAD // _BM,),
    in_specs=[
        pl.BlockSpec((_BM, D), lambda i: (i, 0)),
        pl.BlockSpec((D, D), lambda i: (0, 0)),
        pl.BlockSpec((_BM, 1), lambda i: (i, 0)),
        pl.BlockSpec((_BM, 1), lambda i: (i, 0)),
    ],
    out_specs=[
        pl.BlockSpec((_BM, D), lambda i: (i, 0)),
        pl.BlockSpec((_BM, D), lambda i: (i, 0)),
    ],
    out_shape=[
        jax.ShapeDtypeStruct((NPAD, D), jnp.float32),
        jax.ShapeDtypeStruct((NPAD, D), jnp.float32),
    ],
)


def _tc_comb_body(p0_ref, p1_ref, ri_ref, d0_ref, d1_ref, h_ref, g_ref):
    deg = jnp.maximum(d0_ref[...] + d1_ref[...], 1.0)
    norm = lax.rsqrt(deg)
    h = norm * (p0_ref[...] + p1_ref[...]) + ri_ref[...]
    h_ref[...] = h
    g_ref[...] = h * norm


_tc_comb = pl.pallas_call(
    _tc_comb_body,
    grid=(NPAD // _BM,),
    in_specs=[
        pl.BlockSpec((_BM, D), lambda i: (i, 0)),
        pl.BlockSpec((_BM, D), lambda i: (i, 0)),
        pl.BlockSpec((_BM, D), lambda i: (i, 0)),
        pl.BlockSpec((_BM, 1), lambda i: (i, 0)),
        pl.BlockSpec((_BM, 1), lambda i: (i, 0)),
    ],
    out_specs=[
        pl.BlockSpec((_BM, D), lambda i: (i, 0)),
        pl.BlockSpec((_BM, D), lambda i: (i, 0)),
    ],
    out_shape=[
        jax.ShapeDtypeStruct((NPAD, D), jnp.float32),
        jax.ShapeDtypeStruct((NPAD, D), jnp.float32),
    ],
)


def kernel(features, edge_index, W):
    src = edge_index[0].astype(jnp.int32)
    dst = edge_index[1].astype(jnp.int32)
    pad_e = EPAD - E
    pad_idx = jnp.full((pad_e,), NPAD - 1, dtype=jnp.int32)
    srcp = jnp.concatenate([src, pad_idx])
    dstp = jnp.concatenate([dst, pad_idx])
    xp = jnp.zeros((NPAD, D), jnp.float32).at[:N].set(features)
    zer1 = jnp.zeros((RPT,), jnp.float32)
    ones1 = jnp.ones((EC,), jnp.float32)
    zrows = jnp.zeros((RPT, D), jnp.float32)

    degs = _sc_degs(dstp, zer1, ones1).reshape(NC, NPAD, 1)
    g0, ri = _tc_prep(xp, W, degs[0], degs[1])
    p = _sc_prop(g0, srcp, dstp, zrows).reshape(NC, NPAD, D)
    h1, g1 = _tc_comb(p[0], p[1], ri, degs[0], degs[1])
    p2 = _sc_prop(g1, srcp, dstp, zrows).reshape(NC, NPAD, D)
    h2, _ = _tc_comb(p2[0], p2[1], ri, degs[0], degs[1])
    return h2[:N]
